# baseline (speedup 1.0000x reference)
import jax
import jax.numpy as jnp
from jax.experimental import pallas as pl
from jax.experimental.pallas import tpu as pltpu

_PACK = 16
_TILE_M = 2048
_LANES = 128


def _fused_body(x_ref, z_ref, wx_ref, wz_ref, b_ref, o_ref):
    xm = x_ref[...].reshape(_TILE_M, _LANES)
    zm = z_ref[...].reshape(_TILE_M, _LANES)
    out = (
        jnp.dot(xm, wx_ref[...], preferred_element_type=jnp.float32)
        + jnp.dot(zm, wz_ref[...], preferred_element_type=jnp.float32)
        + b_ref[...]
    )
    o_ref[...] = out.reshape(_TILE_M * _LANES)


def kernel(x, z, wxt, wzt, bt):
    B, depth = x.shape
    pack = _PACK
    lanes = pack * depth

    eye = jnp.eye(pack, dtype=jnp.float32)
    wx_big = jnp.kron(eye, wxt.T)
    wz_big = jnp.kron(eye, wzt.T)
    b_big = jnp.tile(bt.reshape(1, depth), (1, pack))

    n = B * depth
    chunk = _TILE_M * lanes
    flat_x = x.reshape(n)
    flat_z = z.reshape(n)

    grid = (pl.cdiv(n, chunk),)
    vec_spec = pl.BlockSpec((chunk,), lambda i: (i,))
    w_spec = pl.BlockSpec((lanes, lanes), lambda i: (0, 0))
    b_spec = pl.BlockSpec((1, lanes), lambda i: (0, 0))

    out_flat = pl.pallas_call(
        _fused_body,
        out_shape=jax.ShapeDtypeStruct((n,), jnp.float32),
        grid=grid,
        in_specs=[vec_spec, vec_spec, w_spec, w_spec, b_spec],
        out_specs=vec_spec,
        compiler_params=pltpu.CompilerParams(dimension_semantics=("parallel",)),
    )(flat_x, flat_z, wx_big, wz_big, b_big)

    return out_flat.reshape(B, depth)


# speedup vs baseline: 20.0315x; 20.0315x over previous
import jax
import jax.numpy as jnp
from jax.experimental import pallas as pl
from jax.experimental.pallas import tpu as pltpu

_TILE_N = 16384


def _body(xt_ref, zt_ref, w_ref, b_ref, o_ref):
    xz = jnp.concatenate([xt_ref[...], zt_ref[...]], axis=0)
    o_ref[...] = (
        jnp.dot(w_ref[...], xz, preferred_element_type=jnp.float32)
        + b_ref[...]
    )


def kernel(x, z, wxt, wzt, bt):
    B, depth = x.shape
    xt = x.T
    zt = z.T
    w_cat = jnp.concatenate([wxt, wzt], axis=1)

    grid = (pl.cdiv(B, _TILE_N),)
    in_spec = pl.BlockSpec((depth, _TILE_N), lambda i: (0, i))
    w_spec = pl.BlockSpec((depth, 2 * depth), lambda i: (0, 0))
    b_spec = pl.BlockSpec((depth, 1), lambda i: (0, 0))

    out_t = pl.pallas_call(
        _body,
        out_shape=jax.ShapeDtypeStruct((depth, B), jnp.float32),
        grid=grid,
        in_specs=[in_spec, in_spec, w_spec, b_spec],
        out_specs=in_spec,
        compiler_params=pltpu.CompilerParams(dimension_semantics=("parallel",)),
    )(xt, zt, w_cat, bt)

    return out_t.T


# speedup vs baseline: 27.5222x; 1.3739x over previous
import jax
import jax.numpy as jnp
from jax.experimental import pallas as pl
from jax.experimental.pallas import tpu as pltpu

_TILE_N = 32768


def _body(xt_ref, zt_ref, w_ref, b_ref, o_ref):
    xz = jnp.concatenate([xt_ref[...], zt_ref[...]], axis=0)
    o_ref[...] = (
        jnp.dot(w_ref[...], xz, preferred_element_type=jnp.float32)
        + b_ref[...]
    )


def kernel(x, z, wxt, wzt, bt):
    B, depth = x.shape
    xt = x.T
    zt = z.T
    w_cat = jnp.concatenate([wxt, wzt], axis=1)

    grid = (pl.cdiv(B, _TILE_N),)
    in_spec = pl.BlockSpec((depth, _TILE_N), lambda i: (0, i))
    w_spec = pl.BlockSpec((depth, 2 * depth), lambda i: (0, 0))
    b_spec = pl.BlockSpec((depth, 1), lambda i: (0, 0))

    out_t = pl.pallas_call(
        _body,
        out_shape=jax.ShapeDtypeStruct((depth, B), jnp.float32),
        grid=grid,
        in_specs=[in_spec, in_spec, w_spec, b_spec],
        out_specs=in_spec,
        compiler_params=pltpu.CompilerParams(dimension_semantics=("parallel",)),
    )(xt, zt, w_cat, bt)

    return out_t.T


# speedup vs baseline: 32.0169x; 1.1633x over previous
import jax
import jax.numpy as jnp
from jax.experimental import pallas as pl
from jax.experimental.pallas import tpu as pltpu

_TILE_N = 65536


def _body(xt_ref, zt_ref, w_ref, b_ref, o_ref):
    xz = jnp.concatenate([xt_ref[...], zt_ref[...]], axis=0)
    o_ref[...] = (
        jnp.dot(w_ref[...], xz, preferred_element_type=jnp.float32)
        + b_ref[...]
    )


def kernel(x, z, wxt, wzt, bt):
    B, depth = x.shape
    xt = x.T
    zt = z.T
    w_cat = jnp.concatenate([wxt, wzt], axis=1)

    grid = (pl.cdiv(B, _TILE_N),)
    in_spec = pl.BlockSpec((depth, _TILE_N), lambda i: (0, i))
    w_spec = pl.BlockSpec((depth, 2 * depth), lambda i: (0, 0))
    b_spec = pl.BlockSpec((depth, 1), lambda i: (0, 0))

    out_t = pl.pallas_call(
        _body,
        out_shape=jax.ShapeDtypeStruct((depth, B), jnp.float32),
        grid=grid,
        in_specs=[in_spec, in_spec, w_spec, b_spec],
        out_specs=in_spec,
        compiler_params=pltpu.CompilerParams(dimension_semantics=("parallel",)),
    )(xt, zt, w_cat, bt)

    return out_t.T


# speedup vs baseline: 33.2978x; 1.0400x over previous
import jax
import jax.numpy as jnp
from jax.experimental import pallas as pl
from jax.experimental.pallas import tpu as pltpu

_TILE_N = 131072


def _body(xt_ref, zt_ref, w_ref, b_ref, o_ref):
    xz = jnp.concatenate([xt_ref[...], zt_ref[...]], axis=0)
    o_ref[...] = (
        jnp.dot(w_ref[...], xz, preferred_element_type=jnp.float32)
        + b_ref[...]
    )


def kernel(x, z, wxt, wzt, bt):
    B, depth = x.shape
    xt = x.T
    zt = z.T
    w_cat = jnp.concatenate([wxt, wzt], axis=1)

    grid = (pl.cdiv(B, _TILE_N),)
    in_spec = pl.BlockSpec((depth, _TILE_N), lambda i: (0, i))
    w_spec = pl.BlockSpec((depth, 2 * depth), lambda i: (0, 0))
    b_spec = pl.BlockSpec((depth, 1), lambda i: (0, 0))

    out_t = pl.pallas_call(
        _body,
        out_shape=jax.ShapeDtypeStruct((depth, B), jnp.float32),
        grid=grid,
        in_specs=[in_spec, in_spec, w_spec, b_spec],
        out_specs=in_spec,
        compiler_params=pltpu.CompilerParams(dimension_semantics=("parallel",)),
    )(xt, zt, w_cat, bt)

    return out_t.T
